# baseline (speedup 1.0000x reference)
import functools

import jax
import jax.numpy as jnp
from jax import lax
from jax.experimental import pallas as pl
from jax.experimental.pallas import tpu as pltpu
from jax.experimental.pallas import tpu_sc as plsc

_B, _H, _W, _C = 4, 192, 192, 96
_EPP = _H * _W
_HALF = 384 * 192
_ROWS = 48
_WSZ = _ROWS * _W
_NWIN = _H // _ROWS
_CPW = 12
_NPASS = 2 * _CPW
_L = 16
_VPR = _W // _L

_i32 = jnp.int32
_mesh = plsc.VectorSubcoreMesh(core_axis_name="c", subcore_axis_name="s")


@functools.partial(
    pl.kernel,
    mesh=_mesh,
    out_type=jax.ShapeDtypeStruct((_B, 384, _C, 384), jnp.float32),
    scratch_types=[
        pltpu.VMEM((2, _ROWS, _W), jnp.int32),
        pltpu.VMEM((2, _ROWS, _W), jnp.float32),
        pltpu.VMEM((576, 128), jnp.float32),
        pltpu.SemaphoreType.DMA,
        pltpu.SemaphoreType.DMA,
        pltpu.SemaphoreType.DMA,
        pltpu.SemaphoreType.DMA,
        pltpu.SemaphoreType.DMA,
    ],
    compiler_params=pltpu.CompilerParams(
        needs_layout_passes=False, use_tc_tiling_on_sc=True),
)
def _unpool_sc(val_hbm, idx_hbm, out_hbm, idx_v, val_v, acc_v,
               sem_i0, sem_i1, sem_v0, sem_v1, sem_o):
    wid = (lax.axis_index("s") * 2 + lax.axis_index("c")).astype(jnp.int32)
    b = lax.shift_right_logical(wid, _i32(3))
    c0 = (wid & _i32(7)) * _i32(_CPW)
    zeros = jnp.zeros((_L,), jnp.float32)
    third = jnp.float32(0.33333334)
    sem_i = (sem_i0, sem_i1)
    sem_v = (sem_v0, sem_v1)

    def start_in(c, w, buf):
        rows = pl.ds(w * _i32(_ROWS), _ROWS)
        pltpu.async_copy(idx_hbm.at[b, c, rows, :], idx_v.at[_i32(buf)],
                         sem_i[buf])
        pltpu.async_copy(val_hbm.at[b, c, rows, :], val_v.at[_i32(buf)],
                         sem_v[buf])

    def wait_in(c, buf):
        rows = pl.ds(_i32(0), _ROWS)
        pltpu.make_async_copy(
            idx_hbm.at[b, c, rows, :], idx_v.at[_i32(buf)], sem_i[buf]).wait()
        pltpu.make_async_copy(
            val_hbm.at[b, c, rows, :], val_v.at[_i32(buf)], sem_v[buf]).wait()

    def out_slice(c, h):
        return out_hbm.at[b, pl.ds(h * _i32(192), 192), c, :]

    def acc_src():
        return acc_v.at[:, :].reshape(192, 384)

    def pass_body(k, carry):
        c = c0 + lax.shift_right_logical(k, _i32(1))
        h = k & _i32(1)
        lo = h * _i32(_HALF)

        start_in(c, _i32(0), 0)

        @pl.when(k > _i32(0))
        def _():
            pltpu.make_async_copy(acc_src(), out_slice(c, h), sem_o).wait()

        def zero_body(r, carry):
            for u in range(128 // _L):
                acc_v[r, pl.ds(u * _L, _L)] = zeros
            return carry

        lax.fori_loop(_i32(0), _i32(576), zero_body, _i32(0))

        def vec_window(buf):
            def vec_body(r, carry):
                for u in range(_VPR):
                    s = pl.ds(u * _L, _L)
                    a = idx_v[_i32(buf), r, s]
                    t = lax.shift_right_logical(a, _i32(5))
                    q = (t.astype(jnp.float32) * third).astype(jnp.int32)
                    loc = q - lo
                    mask = plsc.bitcast(loc, jnp.uint32) < jnp.uint32(_HALF)
                    row = lax.shift_right_logical(loc, _i32(7))
                    col = loc & _i32(127)
                    v = val_v[_i32(buf), r, s]
                    plsc.addupdate_scatter(acc_v, [row, col], v, mask=mask)
                return carry

            lax.fori_loop(_i32(0), _i32(_ROWS), vec_body, _i32(0))

        start_in(c, _i32(1), 1)
        wait_in(c, 0)
        vec_window(0)
        start_in(c, _i32(2), 0)
        wait_in(c, 1)
        vec_window(1)
        start_in(c, _i32(3), 1)
        wait_in(c, 0)
        vec_window(0)
        wait_in(c, 1)
        vec_window(1)

        pltpu.async_copy(acc_src(), out_slice(c, h), sem_o)
        return carry

    lax.fori_loop(_i32(0), _i32(_NPASS), pass_body, _i32(0))
    pltpu.make_async_copy(acc_src(), out_slice(c0, _i32(1)), sem_o).wait()


def kernel(inputs, pooling_argmax):
    idx32 = pooling_argmax.astype(jnp.int32)
    val_t = jnp.transpose(inputs, (0, 3, 1, 2))
    idx_t = jnp.transpose(idx32, (0, 3, 1, 2))
    out = _unpool_sc(val_t, idx_t)
    return jnp.transpose(out, (0, 1, 3, 2))
